# baseline (speedup 1.0000x reference)
import functools

import jax
import jax.numpy as jnp
from jax import lax
from jax.experimental import pallas as pl
from jax.experimental.pallas import tpu as pltpu
from jax.experimental.pallas import tpu_sc as plsc

_ALPHA = 4
_NW = 32


def _make_sc_pack(C, T, H, W, dtype):
    n = T // _ALPHA
    rows_fast = C * T
    rows_slow = C * n
    fpw = rows_fast // _NW
    mesh = plsc.VectorSubcoreMesh(core_axis_name="c", subcore_axis_name="s")

    @functools.partial(
        pl.kernel,
        mesh=mesh,
        out_type=(
            jax.ShapeDtypeStruct((rows_slow, H, W), dtype),
            jax.ShapeDtypeStruct((rows_fast, H, W), dtype),
        ),
        scratch_types=[pltpu.SemaphoreType.DMA, pltpu.SemaphoreType.DMA],
    )
    def k(frames_hbm, slow_hbm, fast_hbm, sem_f, sem_s):
        wid = lax.axis_index("s") * 2 + lax.axis_index("c")
        cp_f = pltpu.make_async_copy(
            frames_hbm.at[pl.ds(wid * fpw, fpw)],
            fast_hbm.at[pl.ds(wid * fpw, fpw)],
            sem_f,
        )
        cp_f.start()
        j1 = wid
        src1 = (j1 // n) * T + (21 * (j1 % n)) // 5
        cp_s1 = pltpu.make_async_copy(
            frames_hbm.at[pl.ds(src1, 1)], slow_hbm.at[pl.ds(j1, 1)], sem_s
        )
        cp_s1.start()
        j2 = _NW + wid
        src2 = (j2 // n) * T + (21 * (j2 % n)) // 5

        @pl.when(j2 < rows_slow)
        def _():
            cp_s2 = pltpu.make_async_copy(
                frames_hbm.at[pl.ds(src2, 1)], slow_hbm.at[pl.ds(j2, 1)], sem_s
            )
            cp_s2.start()
            cp_s2.wait()

        cp_s1.wait()
        cp_f.wait()

    return k


def kernel(frames):
    C, T, H, W = frames.shape
    n = T // _ALPHA
    frames3 = frames.reshape(C * T, H, W)
    slow3, fast3 = _make_sc_pack(C, T, H, W, frames.dtype)(frames3)
    return (slow3.reshape(C, n, H, W), fast3.reshape(C, T, H, W))


# speedup vs baseline: 40.6823x; 40.6823x over previous
import functools

import jax
import jax.numpy as jnp
from jax import lax
from jax.experimental import pallas as pl
from jax.experimental.pallas import tpu as pltpu
from jax.experimental.pallas import tpu_sc as plsc

_ALPHA = 4
_NW = 32
_QROWS = 96


def _make_sc_gather(C, T, H, W, dtype):
    n = T // _ALPHA
    n_sel = C * n
    qpf = H // _QROWS
    nq = n_sel * qpf
    qpw = nq // _NW
    mesh = plsc.VectorSubcoreMesh(core_axis_name="c", subcore_axis_name="s")

    @functools.partial(
        pl.kernel,
        mesh=mesh,
        out_type=jax.ShapeDtypeStruct((n_sel * H, W), dtype),
        scratch_types=[
            pltpu.VMEM((_QROWS, W), dtype),
            pltpu.VMEM((_QROWS, W), dtype),
            pltpu.SemaphoreType.DMA,
            pltpu.SemaphoreType.DMA,
            pltpu.SemaphoreType.DMA,
            pltpu.SemaphoreType.DMA,
        ],
    )
    def k(table_hbm, out_hbm, buf0, buf1, gs0, gs1, ss0, ss1):
        wid = lax.axis_index("s") * 2 + lax.axis_index("c")
        bufs = (buf0, buf1)
        gsems = (gs0, gs1)
        ssems = (ss0, ss1)

        def src_off(q):
            sel = q // qpf
            quarter = q % qpf
            frame = (sel // n) * T + (21 * (sel % n)) // 5
            return frame * H + quarter * _QROWS

        def gather(q, slot):
            return pltpu.make_async_copy(
                table_hbm.at[pl.ds(src_off(q), _QROWS)], bufs[slot], gsems[slot]
            )

        def scatter(q, slot):
            return pltpu.make_async_copy(
                bufs[slot], out_hbm.at[pl.ds(q * _QROWS, _QROWS)], ssems[slot]
            )

        q0 = wid * qpw
        gather(q0, 0).start()
        for b in range(qpw):
            slot = b % 2
            q = q0 + b
            gather(q, slot).wait()
            scatter(q, slot).start()
            if b + 1 < qpw:
                nslot = (b + 1) % 2
                if b >= 1:
                    scatter(q - 1, nslot).wait()
                gather(q + 1, nslot).start()
        scatter(q0 + qpw - 2, (qpw - 2) % 2).wait()
        scatter(q0 + qpw - 1, (qpw - 1) % 2).wait()

    return k


def _copy_body(in_ref, out_ref):
    out_ref[...] = in_ref[...]


def _tc_copy(frames):
    C, T, H, W = frames.shape
    tb = 8
    return pl.pallas_call(
        _copy_body,
        grid=(C, T // tb),
        in_specs=[pl.BlockSpec((1, tb, H, W), lambda c, t: (c, t, 0, 0))],
        out_specs=pl.BlockSpec((1, tb, H, W), lambda c, t: (c, t, 0, 0)),
        out_shape=jax.ShapeDtypeStruct((C, T, H, W), frames.dtype),
    )(frames)


def kernel(frames):
    C, T, H, W = frames.shape
    n = T // _ALPHA
    table = frames.reshape(C * T * H, W)
    slow2d = _make_sc_gather(C, T, H, W, frames.dtype)(table)
    return (slow2d.reshape(C, n, H, W), _tc_copy(frames))


# speedup vs baseline: 41.0350x; 1.0087x over previous
import functools

import jax
import jax.numpy as jnp
from jax import lax
from jax.experimental import pallas as pl
from jax.experimental.pallas import tpu as pltpu
from jax.experimental.pallas import tpu_sc as plsc

_ALPHA = 4
_NW = 32
_QROWS = 96


def _make_sc_gather(C, T, H, W, dtype):
    n = T // _ALPHA
    n_sel = C * n
    qpf = H // _QROWS
    nq = n_sel * qpf
    qpw = nq // _NW
    mesh = plsc.VectorSubcoreMesh(core_axis_name="c", subcore_axis_name="s")

    @functools.partial(
        pl.kernel,
        mesh=mesh,
        out_type=jax.ShapeDtypeStruct((n_sel * H, W), dtype),
        scratch_types=[
            pltpu.VMEM((_QROWS, W), dtype),
            pltpu.VMEM((_QROWS, W), dtype),
            pltpu.SemaphoreType.DMA,
            pltpu.SemaphoreType.DMA,
            pltpu.SemaphoreType.DMA,
            pltpu.SemaphoreType.DMA,
        ],
    )
    def k(table_hbm, out_hbm, buf0, buf1, gs0, gs1, ss0, ss1):
        wid = lax.axis_index("s") * 2 + lax.axis_index("c")
        bufs = (buf0, buf1)
        gsems = (gs0, gs1)
        ssems = (ss0, ss1)

        def src_off(q):
            sel = q // qpf
            quarter = q % qpf
            frame = (sel // n) * T + (21 * (sel % n)) // 5
            return frame * H + quarter * _QROWS

        def gather(q, slot):
            return pltpu.make_async_copy(
                table_hbm.at[pl.ds(src_off(q), _QROWS)], bufs[slot], gsems[slot]
            )

        def scatter(q, slot):
            return pltpu.make_async_copy(
                bufs[slot], out_hbm.at[pl.ds(q * _QROWS, _QROWS)], ssems[slot]
            )

        q0 = wid * qpw
        gather(q0, 0).start()
        for b in range(qpw):
            slot = b % 2
            q = q0 + b
            gather(q, slot).wait()
            scatter(q, slot).start()
            if b + 1 < qpw:
                nslot = (b + 1) % 2
                if b >= 1:
                    scatter(q - 1, nslot).wait()
                gather(q + 1, nslot).start()
        scatter(q0 + qpw - 2, (qpw - 2) % 2).wait()
        scatter(q0 + qpw - 1, (qpw - 1) % 2).wait()

    return k


def _copy_body(in_ref, out_ref):
    out_ref[...] = in_ref[...]


def _tc_copy(frames):
    C, T, H, W = frames.shape
    tb = 16
    return pl.pallas_call(
        _copy_body,
        grid=(C, T // tb),
        in_specs=[pl.BlockSpec((1, tb, H, W), lambda c, t: (c, t, 0, 0))],
        out_specs=pl.BlockSpec((1, tb, H, W), lambda c, t: (c, t, 0, 0)),
        out_shape=jax.ShapeDtypeStruct((C, T, H, W), frames.dtype),
    )(frames)


def kernel(frames):
    C, T, H, W = frames.shape
    n = T // _ALPHA
    table = frames.reshape(C * T * H, W)
    slow2d = _make_sc_gather(C, T, H, W, frames.dtype)(table)
    return (slow2d.reshape(C, n, H, W), _tc_copy(frames))
